# Initial kernel scaffold; baseline (speedup 1.0000x reference)
import jax
import jax.numpy as jnp
from jax.experimental import pallas as pl


def kernel(x, table, W, b):
    raise NotImplementedError("write your pallas kernel here")


# speedup vs baseline: 9.0578x; 9.0578x over previous
import functools

import jax
import jax.numpy as jnp
from jax import lax
from jax.experimental import pallas as pl
from jax.experimental.pallas import tpu as pltpu
from jax.experimental.pallas import tpu_sc as plsc

VOCAB = 1000000
EMBED = 32
SEQ = 50
BATCH = 4096
ENC_UNITS = 256

N_ROWS = BATCH * SEQ
NUM_CORES = 2
NUM_SUBCORES = 16
NUM_WORKERS = NUM_CORES * NUM_SUBCORES
ROWS_PER_WORKER = N_ROWS // NUM_WORKERS
CHUNK = 800
NUM_CHUNKS = ROWS_PER_WORKER // CHUNK


def _gather_body(table_hbm, idx_hbm, out_hbm, idx_v, rows_v, sem):
    wid = lax.axis_index("s") * NUM_CORES + lax.axis_index("c")
    base = wid * ROWS_PER_WORKER
    for i in range(NUM_CHUNKS):
        off = base + i * CHUNK
        pltpu.sync_copy(idx_hbm.at[pl.ds(off, CHUNK)], idx_v)
        pltpu.async_copy(table_hbm.at[idx_v], rows_v, sem).wait()
        pltpu.sync_copy(rows_v, out_hbm.at[pl.ds(off, CHUNK)])


_gather = pl.kernel(
    _gather_body,
    out_type=jax.ShapeDtypeStruct((N_ROWS, EMBED), jnp.float32),
    mesh=plsc.VectorSubcoreMesh(
        core_axis_name="c", subcore_axis_name="s",
        num_cores=NUM_CORES, num_subcores=NUM_SUBCORES),
    scratch_types=[
        pltpu.VMEM((CHUNK,), jnp.int32),
        pltpu.VMEM((CHUNK, EMBED), jnp.float32),
        pltpu.SemaphoreType.DMA,
    ],
    compiler_params=pltpu.CompilerParams(use_tc_tiling_on_sc=False),
)


BB = 512


def _mlp_body(flat_ref, w_ref, b_ref, out_ref):
    acc = jnp.dot(flat_ref[...], w_ref[...], preferred_element_type=jnp.float32)
    out_ref[...] = jnp.tanh(acc + b_ref[...])


_mlp = pl.pallas_call(
    _mlp_body,
    grid=(BATCH // BB,),
    in_specs=[
        pl.BlockSpec((BB, SEQ * EMBED), lambda i: (i, 0)),
        pl.BlockSpec((SEQ * EMBED, ENC_UNITS), lambda i: (0, 0)),
        pl.BlockSpec((1, ENC_UNITS), lambda i: (0, 0)),
    ],
    out_specs=pl.BlockSpec((BB, ENC_UNITS), lambda i: (i, 0)),
    out_shape=jax.ShapeDtypeStruct((BATCH, ENC_UNITS), jnp.float32),
)


def kernel(x, table, W, b):
    idx = x.reshape(-1).astype(jnp.int32)
    rows = _gather(table, idx)
    flat = rows.reshape(BATCH, SEQ * EMBED)
    return _mlp(flat, W, b.reshape(1, ENC_UNITS))
